# Initial kernel scaffold; baseline (speedup 1.0000x reference)
import jax
import jax.numpy as jnp
from jax.experimental import pallas as pl


def kernel(x, table):
    raise NotImplementedError("write your pallas kernel here")


# speedup vs baseline: 6.3424x; 6.3424x over previous
import functools

import jax
import jax.numpy as jnp
from jax import lax
from jax.experimental import pallas as pl
from jax.experimental.pallas import tpu as pltpu
from jax.experimental.pallas import tpu_sc as plsc


@functools.cache
def _make_gather(B, D):
    info = plsc.get_sparse_core_info()
    NC, NS = info.num_cores, info.num_subcores
    NW = NC * NS
    assert B % NW == 0
    per_w = B // NW
    C = 2048
    assert per_w % C == 0
    n_chunks = per_w // C

    mesh = plsc.VectorSubcoreMesh(core_axis_name="c", subcore_axis_name="s")

    @functools.partial(
        pl.kernel,
        mesh=mesh,
        out_type=jax.ShapeDtypeStruct((B, D), jnp.float32),
        scratch_types=[
            pltpu.VMEM((C,), jnp.int32),
            pltpu.VMEM((C, D), jnp.float32),
            pltpu.SemaphoreType.DMA,
        ],
        compiler_params=pltpu.CompilerParams(use_tc_tiling_on_sc=False),
    )
    def k(table_hbm, idx_hbm, out_hbm, idx_v, rows_v, sem):
        wid = lax.axis_index("s") * NC + lax.axis_index("c")
        base = wid * per_w

        def step(j, carry):
            off = base + j * C
            pltpu.sync_copy(idx_hbm.at[pl.ds(off, C)], idx_v)
            pltpu.async_copy(table_hbm.at[idx_v], rows_v, sem).wait()
            pltpu.sync_copy(rows_v, out_hbm.at[pl.ds(off, C)])
            return carry

        lax.fori_loop(0, n_chunks, step, 0)

    return k


def kernel(x, table):
    B = x.shape[0] * x.shape[1]
    D = table.shape[1]
    idx = x.reshape(B).astype(jnp.int32)
    out = _make_gather(B, D)(table, idx)
    return out.reshape(x.shape + (D,))


# speedup vs baseline: 6.4723x; 1.0205x over previous
import functools

import jax
import jax.numpy as jnp
from jax import lax
from jax.experimental import pallas as pl
from jax.experimental.pallas import tpu as pltpu
from jax.experimental.pallas import tpu_sc as plsc

_NBUF = 4
_CHUNK = 800


@functools.cache
def _make_gather(B, D):
    info = plsc.get_sparse_core_info()
    NC, NS = info.num_cores, info.num_subcores
    NW = NC * NS
    assert B % NW == 0
    per_w = B // NW
    C = _CHUNK
    assert per_w % (C * _NBUF) == 0
    n_groups = per_w // (C * _NBUF)
    assert n_groups >= 2

    mesh = plsc.VectorSubcoreMesh(core_axis_name="c", subcore_axis_name="s")

    @functools.partial(
        pl.kernel,
        mesh=mesh,
        out_type=jax.ShapeDtypeStruct((B, D), jnp.float32),
        scratch_types=(
            [pltpu.VMEM((_NBUF, C), jnp.int32),
             pltpu.VMEM((_NBUF, C, D), jnp.float32)]
            + [pltpu.SemaphoreType.DMA] * (3 * _NBUF)
        ),
        compiler_params=pltpu.CompilerParams(use_tc_tiling_on_sc=False),
    )
    def k(table_hbm, idx_hbm, out_hbm, idx_v, rows_v, *sems):
        sem_idx = sems[:_NBUF]
        sem_g = sems[_NBUF:2 * _NBUF]
        sem_out = sems[2 * _NBUF:]
        wid = lax.axis_index("s") * NC + lax.axis_index("c")
        base = wid * per_w

        def idx_copy(j, b):
            return pltpu.make_async_copy(
                idx_hbm.at[pl.ds(base + j * C, C)], idx_v.at[b], sem_idx[b])

        def gather_copy(b):
            return pltpu.make_async_copy(
                table_hbm.at[idx_v.at[b]], rows_v.at[b], sem_g[b])

        def out_copy(j, b):
            return pltpu.make_async_copy(
                rows_v.at[b], out_hbm.at[pl.ds(base + j * C, C)], sem_out[b])

        for b in range(_NBUF):
            idx_copy(b, b).start()

        for b in range(_NBUF):
            idx_copy(b, b).wait()
            gather_copy(b).start()
        for b in range(_NBUF):
            gather_copy(b).wait()
            out_copy(b, b).start()
            idx_copy(_NBUF + b, b).start()

        def group(g, carry):
            j0 = g * _NBUF
            for b in range(_NBUF):
                out_copy(j0 - _NBUF + b, b).wait()
                idx_copy(j0 + b, b).wait()
                gather_copy(b).start()
            for b in range(_NBUF):
                gather_copy(b).wait()
                out_copy(j0 + b, b).start()
                idx_copy(j0 + _NBUF + b, b).start()
            return carry

        lax.fori_loop(1, n_groups - 1, group, 0)

        j0 = (n_groups - 1) * _NBUF
        for b in range(_NBUF):
            out_copy(j0 - _NBUF + b, b).wait()
            idx_copy(j0 + b, b).wait()
            gather_copy(b).start()
        for b in range(_NBUF):
            gather_copy(b).wait()
            out_copy(j0 + b, b).start()
        for b in range(_NBUF):
            out_copy(j0 + b, b).wait()

    return k


def kernel(x, table):
    B = x.shape[0] * x.shape[1]
    D = table.shape[1]
    idx = x.reshape(B).astype(jnp.int32)
    out = _make_gather(B, D)(table, idx)
    return out.reshape(x.shape + (D,))
